# Initial kernel scaffold; baseline (speedup 1.0000x reference)
import functools

import jax
import jax.numpy as jnp
from jax import lax
from jax.experimental import pallas as pl
from jax.experimental.pallas import tpu as pltpu
from jax.experimental.pallas import tpu_sc as plsc

N = 10000
E = 320000
D = 128
ATT = 4

NC = 2
NS = 16
L = 16
NW = NC * NS
EW = E // NW
C = 80
NCHUNK = EW // C
RPS = N // NS
ZR = 125

_mesh = plsc.VectorSubcoreMesh(core_axis_name="core", subcore_axis_name="subcore")


def _sc_edge_body(src_hbm, dst_hbm, qk_hbm, v_hbm, acc_hbm, den_hbm,
                  src2, dst2, e2, qkd, qks, vrows, zrow, zden,
                  acc_sh, den_sh):
    c_idx = lax.axis_index("core")
    s_idx = lax.axis_index("subcore")
    wid = s_idx * NC + c_idx

    zero = jnp.zeros((L,), jnp.float32)

    @pl.loop(0, ZR)
    def _(r):
        for j in range(D // L):
            zrow[r, pl.ds(j * L, L)] = zero

    @pl.loop(0, RPS // L + 1)
    def _(i):
        zden[pl.ds(i * L, L)] = zero

    row0 = s_idx * RPS

    @pl.loop(0, RPS // ZR)
    def _(b):
        pltpu.sync_copy(zrow, acc_sh.at[pl.ds(row0 + b * ZR, ZR)])

    pltpu.sync_copy(zden.at[pl.ds(0, RPS)], den_sh.at[pl.ds(row0, RPS)])

    plsc.subcore_barrier()

    pltpu.sync_copy(src_hbm.at[wid], src2)
    pltpu.sync_copy(dst_hbm.at[wid], dst2)

    lanes = lax.iota(jnp.int32, L)

    @pl.loop(0, NCHUNK)
    def _(ci):
        pltpu.sync_copy(qk_hbm.at[dst2.at[ci]], qkd)
        pltpu.sync_copy(qk_hbm.at[src2.at[ci]], qks)
        for g in range(C // L):
            rows16 = lanes + g * L
            logit = jnp.zeros((L,), jnp.float32)
            for j in range(ATT):
                qv = plsc.load_gather(qkd, [rows16, jnp.full((L,), j, jnp.int32)])
                kv = plsc.load_gather(qks, [rows16, jnp.full((L,), ATT + j, jnp.int32)])
                logit = logit + qv * kv
            logit = jnp.where(logit >= 0.0, logit, 0.2 * logit)
            e2[ci, pl.ds(g * L, L)] = jnp.exp(logit)

    @pl.loop(0, NCHUNK)
    def _(ci):
        pltpu.sync_copy(v_hbm.at[src2.at[ci]], vrows)

        @pl.loop(0, C)
        def _(i):
            eb = plsc.load_gather(
                e2, [jnp.full((L,), ci, jnp.int32), jnp.full((L,), i, jnp.int32)])
            for r in range(D // L):
                vrows[i, pl.ds(r * L, L)] = vrows[i, pl.ds(r * L, L)] * eb

        pltpu.sync_copy(vrows, acc_sh.at[dst2.at[ci]], add=True)
        pltpu.sync_copy(e2.at[ci], den_sh.at[dst2.at[ci]], add=True)

    plsc.subcore_barrier()

    pltpu.sync_copy(acc_sh.at[pl.ds(row0, RPS)], acc_hbm.at[c_idx, pl.ds(row0, RPS)])
    pltpu.sync_copy(den_sh.at[pl.ds(row0, RPS)], den_hbm.at[c_idx, pl.ds(row0, RPS)])


_sc_edge = pl.kernel(
    _sc_edge_body,
    out_type=(jax.ShapeDtypeStruct((NC, N, D), jnp.float32),
              jax.ShapeDtypeStruct((NC, N), jnp.float32)),
    mesh=_mesh,
    scratch_types=[
        pltpu.VMEM((NCHUNK, C), jnp.int32),
        pltpu.VMEM((NCHUNK, C), jnp.int32),
        pltpu.VMEM((NCHUNK, C), jnp.float32),
        pltpu.VMEM((C, 2 * ATT), jnp.float32),
        pltpu.VMEM((C, 2 * ATT), jnp.float32),
        pltpu.VMEM((C, D), jnp.float32),
        pltpu.VMEM((ZR, D), jnp.float32),
        pltpu.VMEM((RPS + L,), jnp.float32),
        pltpu.VMEM_SHARED((N, D), jnp.float32),
        pltpu.VMEM_SHARED((N,), jnp.float32),
    ],
)


BN = 1000


def _mm_body(x_ref, wqk_ref, wv_ref, qk_ref, v_ref):
    xb = x_ref[...]
    qk_ref[...] = jnp.dot(xb, wqk_ref[...], preferred_element_type=jnp.float32)
    v_ref[...] = jnp.dot(xb, wv_ref[...], preferred_element_type=jnp.float32)


def _mm(x, wqk, wv):
    return pl.pallas_call(
        _mm_body,
        grid=(N // BN,),
        in_specs=[pl.BlockSpec((BN, D), lambda i: (i, 0)),
                  pl.BlockSpec((D, 2 * ATT), lambda i: (0, 0)),
                  pl.BlockSpec((D, D), lambda i: (0, 0))],
        out_specs=[pl.BlockSpec((BN, 2 * ATT), lambda i: (i, 0)),
                   pl.BlockSpec((BN, D), lambda i: (i, 0))],
        out_shape=(jax.ShapeDtypeStruct((N, 2 * ATT), jnp.float32),
                   jax.ShapeDtypeStruct((N, D), jnp.float32)),
    )(x, wqk, wv)


def _combine_mm_body(acc_ref, den_ref, wqk_ref, wv_ref, qk_ref, v_ref):
    a = acc_ref[0] + acc_ref[1]
    d = den_ref[0] + den_ref[1] + 1e-16
    h = a / d[:, None]
    h = jnp.where(h >= 0.0, h, 0.1 * h)
    qk_ref[...] = jnp.dot(h, wqk_ref[...], preferred_element_type=jnp.float32)
    v_ref[...] = jnp.dot(h, wv_ref[...], preferred_element_type=jnp.float32)


def _combine_mm(acc, den, wqk, wv):
    return pl.pallas_call(
        _combine_mm_body,
        grid=(N // BN,),
        in_specs=[pl.BlockSpec((NC, BN, D), lambda i: (0, i, 0)),
                  pl.BlockSpec((NC, BN), lambda i: (0, i)),
                  pl.BlockSpec((D, 2 * ATT), lambda i: (0, 0)),
                  pl.BlockSpec((D, D), lambda i: (0, 0))],
        out_specs=[pl.BlockSpec((BN, 2 * ATT), lambda i: (i, 0)),
                   pl.BlockSpec((BN, D), lambda i: (i, 0))],
        out_shape=(jax.ShapeDtypeStruct((N, 2 * ATT), jnp.float32),
                   jax.ShapeDtypeStruct((N, D), jnp.float32)),
    )(acc, den, wqk, wv)


def _final_body(acc_ref, den_ref, out_ref):
    a = acc_ref[0] + acc_ref[1]
    d = den_ref[0] + den_ref[1] + 1e-16
    h = a / d[:, None]
    out_ref[...] = jnp.where(h >= 0.0, h, 0.1 * h)


def _final(acc, den):
    return pl.pallas_call(
        _final_body,
        grid=(N // BN,),
        in_specs=[pl.BlockSpec((NC, BN, D), lambda i: (0, i, 0)),
                  pl.BlockSpec((NC, BN), lambda i: (0, i))],
        out_specs=pl.BlockSpec((BN, D), lambda i: (i, 0)),
        out_shape=jax.ShapeDtypeStruct((N, D), jnp.float32),
    )(acc, den)


def kernel(x, edge_index, Wq1, Wk1, Wv1, Wq2, Wk2, Wv2):
    src3 = edge_index[0].reshape(NW, NCHUNK, C)
    dst3 = edge_index[1].reshape(NW, NCHUNK, C)

    wqk1 = jnp.concatenate([Wq1, Wk1], axis=1)
    wqk2 = jnp.concatenate([Wq2, Wk2], axis=1)

    qk1, v1 = _mm(x, wqk1, Wv1)
    acc1, den1 = _sc_edge(src3, dst3, qk1, v1)
    qk2, v2 = _combine_mm(acc1, den1, wqk2, Wv2)
    acc2, den2 = _sc_edge(src3, dst3, qk2, v2)
    return _final(acc2, den2)


# speedup vs baseline: 13.5435x; 13.5435x over previous
import functools

import jax
import jax.numpy as jnp
from jax import lax
from jax.experimental import pallas as pl
from jax.experimental.pallas import tpu as pltpu
from jax.experimental.pallas import tpu_sc as plsc

N = 10000
E = 320000
D = 128
ATT = 4

NC = 2
NS = 16
L = 16
NW = NC * NS
EW = E // NW
C = 80
NCHUNK = EW // C
NP_ = 10240
RPS = NP_ // NS
ZR = 128

_mesh = plsc.VectorSubcoreMesh(core_axis_name="core", subcore_axis_name="subcore")


def _sc_edge_body(src_hbm, dst_hbm, qk_hbm, v_hbm, acc_hbm, den_hbm,
                  src2, dst2, e2, qkd, qks, vrows, zden,
                  acc_sh, den_sh):
    c_idx = lax.axis_index("core")
    s_idx = lax.axis_index("subcore")
    wid = s_idx * NC + c_idx

    zero = jnp.zeros((L,), jnp.float32)

    @pl.loop(0, C)
    def _(r):
        for j in range(D // L):
            vrows[r, pl.ds(j * L, L)] = zero

    @pl.loop(0, RPS // L)
    def _(i):
        zden[pl.ds(i * L, L)] = zero

    row0 = s_idx * RPS

    @pl.loop(0, RPS // C)
    def _(b):
        pltpu.sync_copy(vrows, acc_sh.at[pl.ds(row0 + b * C, C)])

    pltpu.sync_copy(zden, den_sh.at[pl.ds(row0, RPS)])

    plsc.subcore_barrier()

    pltpu.sync_copy(src_hbm.at[wid], src2)
    pltpu.sync_copy(dst_hbm.at[wid], dst2)

    lanes = lax.iota(jnp.int32, L)

    @pl.loop(0, NCHUNK)
    def _(ci):
        pltpu.sync_copy(qk_hbm.at[dst2.at[ci]], qkd)
        pltpu.sync_copy(qk_hbm.at[src2.at[ci]], qks)
        for g in range(C // L):
            rows16 = lanes + g * L
            logit = jnp.zeros((L,), jnp.float32)
            for j in range(ATT):
                qv = plsc.load_gather(qkd, [rows16, jnp.full((L,), j, jnp.int32)])
                kv = plsc.load_gather(qks, [rows16, jnp.full((L,), ATT + j, jnp.int32)])
                logit = logit + qv * kv
            logit = jnp.where(logit >= 0.0, logit, 0.2 * logit)
            e2[ci, pl.ds(g * L, L)] = jnp.exp(logit)

    @pl.loop(0, NCHUNK)
    def _(ci):
        pltpu.sync_copy(v_hbm.at[src2.at[ci]], vrows)

        @pl.loop(0, C)
        def _(i):
            eb = plsc.load_gather(
                e2, [jnp.full((L,), ci, jnp.int32), jnp.full((L,), i, jnp.int32)])
            for r in range(D // L):
                vrows[i, pl.ds(r * L, L)] = vrows[i, pl.ds(r * L, L)] * eb

        pltpu.sync_copy(vrows, acc_sh.at[dst2.at[ci]], add=True)
        pltpu.sync_copy(e2.at[ci], den_sh.at[dst2.at[ci]], add=True)

    plsc.subcore_barrier()

    pltpu.sync_copy(acc_sh.at[pl.ds(row0, RPS)], acc_hbm.at[c_idx, pl.ds(row0, RPS)])
    pltpu.sync_copy(den_sh.at[pl.ds(row0, RPS)], den_hbm.at[c_idx, pl.ds(row0, RPS)])


_sc_edge = pl.kernel(
    _sc_edge_body,
    out_type=(jax.ShapeDtypeStruct((NC, NP_, D), jnp.float32),
              jax.ShapeDtypeStruct((NC, NP_), jnp.float32)),
    mesh=_mesh,
    compiler_params=pltpu.CompilerParams(needs_layout_passes=False,
                                         use_tc_tiling_on_sc=False),
    scratch_types=[
        pltpu.VMEM((NCHUNK, C), jnp.int32),
        pltpu.VMEM((NCHUNK, C), jnp.int32),
        pltpu.VMEM((NCHUNK, C), jnp.float32),
        pltpu.VMEM((C, 2 * ATT), jnp.float32),
        pltpu.VMEM((C, 2 * ATT), jnp.float32),
        pltpu.VMEM((C, D), jnp.float32),
        pltpu.VMEM((RPS,), jnp.float32),
        pltpu.VMEM_SHARED((NP_, D), jnp.float32),
        pltpu.VMEM_SHARED((NP_,), jnp.float32),
    ],
)


BN = 1000


def _mm_body(x_ref, wqk_ref, wv_ref, qk_ref, v_ref):
    xb = x_ref[...]
    qk_ref[...] = jnp.dot(xb, wqk_ref[...], preferred_element_type=jnp.float32)
    v_ref[...] = jnp.dot(xb, wv_ref[...], preferred_element_type=jnp.float32)


def _mm(x, wqk, wv):
    return pl.pallas_call(
        _mm_body,
        grid=(N // BN,),
        in_specs=[pl.BlockSpec((BN, D), lambda i: (i, 0)),
                  pl.BlockSpec((D, 2 * ATT), lambda i: (0, 0)),
                  pl.BlockSpec((D, D), lambda i: (0, 0))],
        out_specs=[pl.BlockSpec((BN, 2 * ATT), lambda i: (i, 0)),
                   pl.BlockSpec((BN, D), lambda i: (i, 0))],
        out_shape=(jax.ShapeDtypeStruct((N, 2 * ATT), jnp.float32),
                   jax.ShapeDtypeStruct((N, D), jnp.float32)),
    )(x, wqk, wv)


def _combine_mm_body(acc_ref, den_ref, wqk_ref, wv_ref, qk_ref, v_ref):
    a = acc_ref[0] + acc_ref[1]
    d = den_ref[0] + den_ref[1] + 1e-16
    h = a / d
    h = jnp.where(h >= 0.0, h, 0.1 * h)
    qk_ref[...] = jnp.dot(h, wqk_ref[...], preferred_element_type=jnp.float32)
    v_ref[...] = jnp.dot(h, wv_ref[...], preferred_element_type=jnp.float32)


def _combine_mm(acc, den, wqk, wv):
    return pl.pallas_call(
        _combine_mm_body,
        grid=(N // BN,),
        in_specs=[pl.BlockSpec((NC, BN, D), lambda i: (0, i, 0)),
                  pl.BlockSpec((NC, BN, 1), lambda i: (0, i, 0)),
                  pl.BlockSpec((D, 2 * ATT), lambda i: (0, 0)),
                  pl.BlockSpec((D, D), lambda i: (0, 0))],
        out_specs=[pl.BlockSpec((BN, 2 * ATT), lambda i: (i, 0)),
                   pl.BlockSpec((BN, D), lambda i: (i, 0))],
        out_shape=(jax.ShapeDtypeStruct((N, 2 * ATT), jnp.float32),
                   jax.ShapeDtypeStruct((N, D), jnp.float32)),
    )(acc, den, wqk, wv)


def _final_body(acc_ref, den_ref, out_ref):
    a = acc_ref[0] + acc_ref[1]
    d = den_ref[0] + den_ref[1] + 1e-16
    h = a / d
    out_ref[...] = jnp.where(h >= 0.0, h, 0.1 * h)


def _final(acc, den):
    return pl.pallas_call(
        _final_body,
        grid=(N // BN,),
        in_specs=[pl.BlockSpec((NC, BN, D), lambda i: (0, i, 0)),
                  pl.BlockSpec((NC, BN, 1), lambda i: (0, i, 0))],
        out_specs=pl.BlockSpec((BN, D), lambda i: (i, 0)),
        out_shape=jax.ShapeDtypeStruct((N, D), jnp.float32),
    )(acc, den)


def kernel(x, edge_index, Wq1, Wk1, Wv1, Wq2, Wk2, Wv2):
    src3 = edge_index[0].reshape(NW, NCHUNK, C)
    dst3 = edge_index[1].reshape(NW, NCHUNK, C)

    wqk1 = jnp.concatenate([Wq1, Wk1], axis=1)
    wqk2 = jnp.concatenate([Wq2, Wk2], axis=1)

    qk1, v1 = _mm(x, wqk1, Wv1)
    acc1, den1 = _sc_edge(src3, dst3, qk1, v1)
    qk2, v2 = _combine_mm(acc1, den1[..., None], wqk2, Wv2)
    acc2, den2 = _sc_edge(src3, dst3, qk2, v2)
    return _final(acc2, den2[..., None])


# speedup vs baseline: 22.3690x; 1.6516x over previous
import functools

import jax
import jax.numpy as jnp
from jax import lax
from jax.experimental import pallas as pl
from jax.experimental.pallas import tpu as pltpu
from jax.experimental.pallas import tpu_sc as plsc

N = 10000
E = 320000
D = 128
ATT = 4

NC = 2
NS = 16
L = 16
NW = NC * NS
EW = E // NW
C = 80
NCHUNK = EW // C
NP_ = 10240
RPS = NP_ // NS
ZR = 128

_mesh = plsc.VectorSubcoreMesh(core_axis_name="core", subcore_axis_name="subcore")


def _sc_edge_body(src_hbm, dst_hbm, qk_hbm, v_hbm, acc_hbm, den_hbm,
                  dst2, sidx, qkd, qks, vrows, ebuf, zden,
                  gsem, isem, ssem, acc_sh, den_sh):
    c_idx = lax.axis_index("core")
    s_idx = lax.axis_index("subcore")
    wid = s_idx * NC + c_idx

    zero = jnp.zeros((L,), jnp.float32)

    @pl.loop(0, C)
    def _(r):
        for j in range(D // L):
            vrows[0][r, pl.ds(j * L, L)] = zero

    @pl.loop(0, RPS // L)
    def _(i):
        zden[pl.ds(i * L, L)] = zero

    row0 = s_idx * RPS

    @pl.loop(0, RPS // C)
    def _(b):
        pltpu.sync_copy(vrows[0], acc_sh.at[pl.ds(row0 + b * C, C)])

    pltpu.sync_copy(zden, den_sh.at[pl.ds(row0, RPS)])

    pltpu.sync_copy(dst_hbm.at[wid], dst2)

    plsc.subcore_barrier()

    lanes = lax.iota(jnp.int32, L)

    def start_gather(ci, b):
        pltpu.async_copy(qk_hbm.at[dst2.at[ci]], qkd[b], gsem[b])
        pltpu.async_copy(qk_hbm.at[sidx[b]], qks[b], gsem[b])
        pltpu.async_copy(v_hbm.at[sidx[b]], vrows[b], gsem[b])

    def wait_gather(b):
        pltpu.make_async_copy(qk_hbm.at[pl.ds(0, C)], qkd[b], gsem[b]).wait()
        pltpu.make_async_copy(qk_hbm.at[pl.ds(0, C)], qks[b], gsem[b]).wait()
        pltpu.make_async_copy(v_hbm.at[pl.ds(0, C)], vrows[b], gsem[b]).wait()

    def wait_scatter(b):
        pltpu.make_async_copy(v_hbm.at[pl.ds(0, C)], vrows[b], ssem[b]).wait()
        pltpu.make_async_copy(den_hbm.at[0, pl.ds(0, C)], ebuf[b], ssem[b]).wait()

    def compute(b):
        for g in range(C // L):
            rows16 = lanes + g * L
            logit = jnp.zeros((L,), jnp.float32)
            for j in range(ATT):
                qv = plsc.load_gather(qkd[b], [rows16, jnp.full((L,), j, jnp.int32)])
                kv = plsc.load_gather(qks[b], [rows16, jnp.full((L,), ATT + j, jnp.int32)])
                logit = logit + qv * kv
            logit = jnp.where(logit >= 0.0, logit, 0.2 * logit)
            ebuf[b][pl.ds(g * L, L)] = jnp.exp(logit)

        @pl.loop(0, C, step=4)
        def _(i0):
            for u in range(4):
                i = i0 + u
                eb = plsc.load_gather(ebuf[b], [jnp.full((L,), i, jnp.int32)])
                for r in range(D // L):
                    vrows[b][i, pl.ds(r * L, L)] = vrows[b][i, pl.ds(r * L, L)] * eb

    def start_scatter(ci, b):
        pltpu.async_copy(vrows[b], acc_sh.at[dst2.at[ci]], ssem[b], add=True)
        pltpu.async_copy(ebuf[b], den_sh.at[dst2.at[ci]], ssem[b], add=True)

    def step(ci, b, first=False):
        bn = 1 - b

        @pl.when(ci + 1 < NCHUNK)
        def _():
            pltpu.async_copy(src_hbm.at[wid, ci + 1], sidx[bn], isem[bn])

        wait_gather(b)
        compute(b)
        start_scatter(ci, b)
        if not first:
            wait_scatter(bn)

        @pl.when(ci + 1 < NCHUNK)
        def _():
            pltpu.make_async_copy(src_hbm.at[wid, 0], sidx[bn], isem[bn]).wait()
            start_gather(ci + 1, bn)

    pltpu.sync_copy(src_hbm.at[wid, 0], sidx[0])
    start_gather(0, 0)
    step(0, 0, first=True)

    @pl.loop(0, (NCHUNK - 1) // 2)
    def _(p):
        step(2 * p + 1, 1)
        step(2 * p + 2, 0)

    wait_scatter(0)

    plsc.subcore_barrier()

    pltpu.sync_copy(acc_sh.at[pl.ds(row0, RPS)], acc_hbm.at[c_idx, pl.ds(row0, RPS)])
    pltpu.sync_copy(den_sh.at[pl.ds(row0, RPS)], den_hbm.at[c_idx, pl.ds(row0, RPS)])


_sc_edge = pl.kernel(
    _sc_edge_body,
    out_type=(jax.ShapeDtypeStruct((NC, NP_, D), jnp.float32),
              jax.ShapeDtypeStruct((NC, NP_), jnp.float32)),
    mesh=_mesh,
    compiler_params=pltpu.CompilerParams(needs_layout_passes=False,
                                         use_tc_tiling_on_sc=False),
    scratch_types=[
        pltpu.VMEM((NCHUNK, C), jnp.int32),
        (pltpu.VMEM((C,), jnp.int32),) * 2,
        (pltpu.VMEM((C, 2 * ATT), jnp.float32),) * 2,
        (pltpu.VMEM((C, 2 * ATT), jnp.float32),) * 2,
        (pltpu.VMEM((C, D), jnp.float32),) * 2,
        (pltpu.VMEM((C,), jnp.float32),) * 2,
        pltpu.VMEM((RPS,), jnp.float32),
        (pltpu.SemaphoreType.DMA,) * 2,
        (pltpu.SemaphoreType.DMA,) * 2,
        (pltpu.SemaphoreType.DMA,) * 2,
        pltpu.VMEM_SHARED((NP_, D), jnp.float32),
        pltpu.VMEM_SHARED((NP_,), jnp.float32),
    ],
)


BN = 1000


def _mm_body(x_ref, wqk_ref, wv_ref, qk_ref, v_ref):
    xb = x_ref[...]
    qk_ref[...] = jnp.dot(xb, wqk_ref[...], preferred_element_type=jnp.float32)
    v_ref[...] = jnp.dot(xb, wv_ref[...], preferred_element_type=jnp.float32)


def _mm(x, wqk, wv):
    return pl.pallas_call(
        _mm_body,
        grid=(N // BN,),
        in_specs=[pl.BlockSpec((BN, D), lambda i: (i, 0)),
                  pl.BlockSpec((D, 2 * ATT), lambda i: (0, 0)),
                  pl.BlockSpec((D, D), lambda i: (0, 0))],
        out_specs=[pl.BlockSpec((BN, 2 * ATT), lambda i: (i, 0)),
                   pl.BlockSpec((BN, D), lambda i: (i, 0))],
        out_shape=(jax.ShapeDtypeStruct((N, 2 * ATT), jnp.float32),
                   jax.ShapeDtypeStruct((N, D), jnp.float32)),
    )(x, wqk, wv)


def _combine_mm_body(acc_ref, den_ref, wqk_ref, wv_ref, qk_ref, v_ref):
    a = acc_ref[0] + acc_ref[1]
    d = den_ref[0] + den_ref[1] + 1e-16
    h = a / d
    h = jnp.where(h >= 0.0, h, 0.1 * h)
    qk_ref[...] = jnp.dot(h, wqk_ref[...], preferred_element_type=jnp.float32)
    v_ref[...] = jnp.dot(h, wv_ref[...], preferred_element_type=jnp.float32)


def _combine_mm(acc, den, wqk, wv):
    return pl.pallas_call(
        _combine_mm_body,
        grid=(N // BN,),
        in_specs=[pl.BlockSpec((NC, BN, D), lambda i: (0, i, 0)),
                  pl.BlockSpec((NC, BN, 1), lambda i: (0, i, 0)),
                  pl.BlockSpec((D, 2 * ATT), lambda i: (0, 0)),
                  pl.BlockSpec((D, D), lambda i: (0, 0))],
        out_specs=[pl.BlockSpec((BN, 2 * ATT), lambda i: (i, 0)),
                   pl.BlockSpec((BN, D), lambda i: (i, 0))],
        out_shape=(jax.ShapeDtypeStruct((N, 2 * ATT), jnp.float32),
                   jax.ShapeDtypeStruct((N, D), jnp.float32)),
    )(acc, den, wqk, wv)


def _final_body(acc_ref, den_ref, out_ref):
    a = acc_ref[0] + acc_ref[1]
    d = den_ref[0] + den_ref[1] + 1e-16
    h = a / d
    out_ref[...] = jnp.where(h >= 0.0, h, 0.1 * h)


def _final(acc, den):
    return pl.pallas_call(
        _final_body,
        grid=(N // BN,),
        in_specs=[pl.BlockSpec((NC, BN, D), lambda i: (0, i, 0)),
                  pl.BlockSpec((NC, BN, 1), lambda i: (0, i, 0))],
        out_specs=pl.BlockSpec((BN, D), lambda i: (i, 0)),
        out_shape=jax.ShapeDtypeStruct((N, D), jnp.float32),
    )(acc, den)


def kernel(x, edge_index, Wq1, Wk1, Wv1, Wq2, Wk2, Wv2):
    src3 = edge_index[0].reshape(NW, NCHUNK, C)
    dst3 = edge_index[1].reshape(NW, NCHUNK, C)

    wqk1 = jnp.concatenate([Wq1, Wk1], axis=1)
    wqk2 = jnp.concatenate([Wq2, Wk2], axis=1)

    qk1, v1 = _mm(x, wqk1, Wv1)
    acc1, den1 = _sc_edge(src3, dst3, qk1, v1)
    qk2, v2 = _combine_mm(acc1, den1[..., None], wqk2, Wv2)
    acc2, den2 = _sc_edge(src3, dst3, qk2, v2)
    return _final(acc2, den2[..., None])


# speedup vs baseline: 22.3823x; 1.0006x over previous
import functools

import jax
import jax.numpy as jnp
from jax import lax
from jax.experimental import pallas as pl
from jax.experimental.pallas import tpu as pltpu
from jax.experimental.pallas import tpu_sc as plsc

N = 10000
E = 320000
D = 128
ATT = 4

NC = 2
NS = 16
L = 16
NW = NC * NS
EW = E // NW
C = 80
NCHUNK = EW // C
NP_ = 10240
RPS = NP_ // NS
ZR = 128

_mesh = plsc.VectorSubcoreMesh(core_axis_name="core", subcore_axis_name="subcore")


def _sc_edge_body(src_hbm, dst_hbm, qk_hbm, v_hbm, acc_hbm, den_hbm,
                  dst2, sidx, qkd, qks, vrows, ebuf, zden,
                  gsem, isem, ssem, acc_sh, den_sh):
    c_idx = lax.axis_index("core")
    s_idx = lax.axis_index("subcore")
    wid = s_idx * NC + c_idx

    zero = jnp.zeros((L,), jnp.float32)

    @pl.loop(0, C)
    def _(r):
        for j in range(D // L):
            vrows[0][r, pl.ds(j * L, L)] = zero

    @pl.loop(0, RPS // L)
    def _(i):
        zden[pl.ds(i * L, L)] = zero

    row0 = s_idx * RPS

    @pl.loop(0, RPS // C)
    def _(b):
        pltpu.sync_copy(vrows[0], acc_sh.at[pl.ds(row0 + b * C, C)])

    pltpu.sync_copy(zden, den_sh.at[pl.ds(row0, RPS)])

    pltpu.sync_copy(dst_hbm.at[wid], dst2)

    plsc.subcore_barrier()

    lanes = lax.iota(jnp.int32, L)

    def start_gather(ci, b):
        pltpu.async_copy(qk_hbm.at[dst2.at[ci]], qkd[b], gsem[b])
        pltpu.async_copy(qk_hbm.at[sidx[b]], qks[b], gsem[b])
        pltpu.async_copy(v_hbm.at[sidx[b]], vrows[b], gsem[b])

    def wait_gather(b):
        pltpu.make_async_copy(qk_hbm.at[pl.ds(0, C)], qkd[b], gsem[b]).wait()
        pltpu.make_async_copy(qk_hbm.at[pl.ds(0, C)], qks[b], gsem[b]).wait()
        pltpu.make_async_copy(v_hbm.at[pl.ds(0, C)], vrows[b], gsem[b]).wait()

    def wait_scatter(b):
        pltpu.make_async_copy(v_hbm.at[pl.ds(0, C)], vrows[b], ssem[b]).wait()
        pltpu.make_async_copy(den_hbm.at[0, pl.ds(0, C)], ebuf[b], ssem[b]).wait()

    def compute(ci, b):
        for g in range(C // L):
            rows16 = lanes + g * L
            logit = jnp.zeros((L,), jnp.float32)
            for j in range(ATT):
                qv = plsc.load_gather(qkd[b], [rows16, jnp.full((L,), j, jnp.int32)])
                kv = plsc.load_gather(qks[b], [rows16, jnp.full((L,), ATT + j, jnp.int32)])
                logit = logit + qv * kv
            logit = jnp.where(logit >= 0.0, logit, 0.2 * logit)
            ebuf[b][pl.ds(g * L, L)] = jnp.exp(logit)

        pltpu.async_copy(ebuf[b], den_sh.at[dst2.at[ci]], ssem[b], add=True)

        @pl.loop(0, C, step=4)
        def _(i0):
            for u in range(4):
                i = i0 + u
                eb = plsc.load_gather(ebuf[b], [jnp.full((L,), i, jnp.int32)])
                for r in range(D // L):
                    vrows[b][i, pl.ds(r * L, L)] = vrows[b][i, pl.ds(r * L, L)] * eb

        pltpu.async_copy(vrows[b], acc_sh.at[dst2.at[ci]], ssem[b], add=True)

    def step(ci, b, first=False):
        bn = 1 - b

        @pl.when(ci + 1 < NCHUNK)
        def _():
            pltpu.async_copy(src_hbm.at[wid, ci + 1], sidx[bn], isem[bn])

        wait_gather(b)
        compute(ci, b)
        if not first:
            wait_scatter(bn)

        @pl.when(ci + 1 < NCHUNK)
        def _():
            pltpu.make_async_copy(src_hbm.at[wid, 0], sidx[bn], isem[bn]).wait()
            start_gather(ci + 1, bn)

    pltpu.sync_copy(src_hbm.at[wid, 0], sidx[0])
    start_gather(0, 0)
    step(0, 0, first=True)

    @pl.loop(0, (NCHUNK - 1) // 2)
    def _(p):
        step(2 * p + 1, 1)
        step(2 * p + 2, 0)

    wait_scatter(0)

    plsc.subcore_barrier()

    pltpu.sync_copy(acc_sh.at[pl.ds(row0, RPS)], acc_hbm.at[c_idx, pl.ds(row0, RPS)])
    pltpu.sync_copy(den_sh.at[pl.ds(row0, RPS)], den_hbm.at[c_idx, pl.ds(row0, RPS)])


_sc_edge = pl.kernel(
    _sc_edge_body,
    out_type=(jax.ShapeDtypeStruct((NC, NP_, D), jnp.float32),
              jax.ShapeDtypeStruct((NC, NP_), jnp.float32)),
    mesh=_mesh,
    compiler_params=pltpu.CompilerParams(needs_layout_passes=False,
                                         use_tc_tiling_on_sc=False),
    scratch_types=[
        pltpu.VMEM((NCHUNK, C), jnp.int32),
        (pltpu.VMEM((C,), jnp.int32),) * 2,
        (pltpu.VMEM((C, 2 * ATT), jnp.float32),) * 2,
        (pltpu.VMEM((C, 2 * ATT), jnp.float32),) * 2,
        (pltpu.VMEM((C, D), jnp.float32),) * 2,
        (pltpu.VMEM((C,), jnp.float32),) * 2,
        pltpu.VMEM((RPS,), jnp.float32),
        (pltpu.SemaphoreType.DMA,) * 2,
        (pltpu.SemaphoreType.DMA,) * 2,
        (pltpu.SemaphoreType.DMA,) * 2,
        pltpu.VMEM_SHARED((NP_, D), jnp.float32),
        pltpu.VMEM_SHARED((NP_,), jnp.float32),
    ],
)


BN = 1000


def _mm_body(x_ref, wqk_ref, wv_ref, qk_ref, v_ref):
    xb = x_ref[...]
    qk_ref[...] = jnp.dot(xb, wqk_ref[...], preferred_element_type=jnp.float32)
    v_ref[...] = jnp.dot(xb, wv_ref[...], preferred_element_type=jnp.float32)


def _mm(x, wqk, wv):
    return pl.pallas_call(
        _mm_body,
        grid=(N // BN,),
        in_specs=[pl.BlockSpec((BN, D), lambda i: (i, 0)),
                  pl.BlockSpec((D, 2 * ATT), lambda i: (0, 0)),
                  pl.BlockSpec((D, D), lambda i: (0, 0))],
        out_specs=[pl.BlockSpec((BN, 2 * ATT), lambda i: (i, 0)),
                   pl.BlockSpec((BN, D), lambda i: (i, 0))],
        out_shape=(jax.ShapeDtypeStruct((N, 2 * ATT), jnp.float32),
                   jax.ShapeDtypeStruct((N, D), jnp.float32)),
    )(x, wqk, wv)


def _combine_mm_body(acc_ref, den_ref, wqk_ref, wv_ref, qk_ref, v_ref):
    a = acc_ref[0] + acc_ref[1]
    d = den_ref[0] + den_ref[1] + 1e-16
    h = a / d
    h = jnp.where(h >= 0.0, h, 0.1 * h)
    qk_ref[...] = jnp.dot(h, wqk_ref[...], preferred_element_type=jnp.float32)
    v_ref[...] = jnp.dot(h, wv_ref[...], preferred_element_type=jnp.float32)


def _combine_mm(acc, den, wqk, wv):
    return pl.pallas_call(
        _combine_mm_body,
        grid=(N // BN,),
        in_specs=[pl.BlockSpec((NC, BN, D), lambda i: (0, i, 0)),
                  pl.BlockSpec((NC, BN, 1), lambda i: (0, i, 0)),
                  pl.BlockSpec((D, 2 * ATT), lambda i: (0, 0)),
                  pl.BlockSpec((D, D), lambda i: (0, 0))],
        out_specs=[pl.BlockSpec((BN, 2 * ATT), lambda i: (i, 0)),
                   pl.BlockSpec((BN, D), lambda i: (i, 0))],
        out_shape=(jax.ShapeDtypeStruct((N, 2 * ATT), jnp.float32),
                   jax.ShapeDtypeStruct((N, D), jnp.float32)),
    )(acc, den, wqk, wv)


def _final_body(acc_ref, den_ref, out_ref):
    a = acc_ref[0] + acc_ref[1]
    d = den_ref[0] + den_ref[1] + 1e-16
    h = a / d
    out_ref[...] = jnp.where(h >= 0.0, h, 0.1 * h)


def _final(acc, den):
    return pl.pallas_call(
        _final_body,
        grid=(N // BN,),
        in_specs=[pl.BlockSpec((NC, BN, D), lambda i: (0, i, 0)),
                  pl.BlockSpec((NC, BN, 1), lambda i: (0, i, 0))],
        out_specs=pl.BlockSpec((BN, D), lambda i: (i, 0)),
        out_shape=jax.ShapeDtypeStruct((N, D), jnp.float32),
    )(acc, den)


def kernel(x, edge_index, Wq1, Wk1, Wv1, Wq2, Wk2, Wv2):
    src3 = edge_index[0].reshape(NW, NCHUNK, C)
    dst3 = edge_index[1].reshape(NW, NCHUNK, C)

    wqk1 = jnp.concatenate([Wq1, Wk1], axis=1)
    wqk2 = jnp.concatenate([Wq2, Wk2], axis=1)

    qk1, v1 = _mm(x, wqk1, Wv1)
    acc1, den1 = _sc_edge(src3, dst3, qk1, v1)
    qk2, v2 = _combine_mm(acc1, den1[..., None], wqk2, Wv2)
    acc2, den2 = _sc_edge(src3, dst3, qk2, v2)
    return _final(acc2, den2[..., None])


# speedup vs baseline: 35.7945x; 1.5992x over previous
import functools

import jax
import jax.numpy as jnp
from jax import lax
from jax.experimental import pallas as pl
from jax.experimental.pallas import tpu as pltpu
from jax.experimental.pallas import tpu_sc as plsc

N = 10000
E = 320000
D = 128
ATT = 4

NC = 2
NS = 16
L = 16
NW = NC * NS
EW = E // NW
C = 80
NCHUNK = EW // C
NP_ = 10240
RPS = NP_ // NS
ZR = 128

_mesh = plsc.VectorSubcoreMesh(core_axis_name="core", subcore_axis_name="subcore")


def _sc_edge_body(src_hbm, dst_hbm, qk_hbm, v_hbm, acc_hbm, den_hbm,
                  dst2, sidx, qkd, qks, vrows, ebuf, zden,
                  gsem, isem, ssem, acc_sh, den_sh):
    c_idx = lax.axis_index("core")
    s_idx = lax.axis_index("subcore")
    wid = s_idx * NC + c_idx

    zero = jnp.zeros((L,), jnp.float32)

    @pl.loop(0, C)
    def _(r):
        for j in range(D // L):
            vrows[0][r, pl.ds(j * L, L)] = zero

    @pl.loop(0, RPS // L)
    def _(i):
        zden[pl.ds(i * L, L)] = zero

    row0 = s_idx * RPS

    @pl.loop(0, RPS // C)
    def _(b):
        pltpu.sync_copy(vrows[0], acc_sh.at[pl.ds(row0 + b * C, C)])

    pltpu.sync_copy(zden, den_sh.at[pl.ds(row0, RPS)])

    pltpu.sync_copy(dst_hbm.at[wid], dst2)

    plsc.subcore_barrier()

    lanes = lax.iota(jnp.int32, L)

    def start_gather(ci, b):
        pltpu.async_copy(qk_hbm.at[dst2.at[ci]], qkd[b], gsem[b])
        pltpu.async_copy(qk_hbm.at[sidx[b]], qks[b], gsem[b])
        pltpu.async_copy(v_hbm.at[sidx[b]], vrows[b], gsem[b])

    def wait_gather(b):
        pltpu.make_async_copy(qk_hbm.at[pl.ds(0, C)], qkd[b], gsem[b]).wait()
        pltpu.make_async_copy(qk_hbm.at[pl.ds(0, C)], qks[b], gsem[b]).wait()
        pltpu.make_async_copy(v_hbm.at[pl.ds(0, C)], vrows[b], gsem[b]).wait()

    def wait_scatter(b):
        pltpu.make_async_copy(v_hbm.at[pl.ds(0, C)], vrows[b], ssem[b]).wait()
        pltpu.make_async_copy(den_hbm.at[0, pl.ds(0, C)], ebuf[b], ssem[b]).wait()

    def compute(ci, b):
        for g in range(C // L):
            rows16 = lanes + g * L
            logit = jnp.zeros((L,), jnp.float32)
            for j in range(ATT):
                qv = plsc.load_gather(qkd[b], [rows16, jnp.full((L,), j, jnp.int32)])
                kv = plsc.load_gather(qks[b], [rows16, jnp.full((L,), ATT + j, jnp.int32)])
                logit = logit + qv * kv
            logit = jnp.where(logit >= 0.0, logit, 0.2 * logit)
            ebuf[b][pl.ds(g * L, L)] = jnp.exp(logit)

        pltpu.async_copy(ebuf[b], den_sh.at[dst2.at[ci]], ssem[b], add=True)

        @pl.loop(0, C, step=4)
        def _(i0):
            for u in range(4):
                i = i0 + u
                eb = plsc.load_gather(ebuf[b], [jnp.full((L,), i, jnp.int32)])
                for r in range(D // L):
                    vrows[b][i, pl.ds(r * L, L)] = vrows[b][i, pl.ds(r * L, L)] * eb

        pltpu.async_copy(vrows[b], acc_sh.at[dst2.at[ci]], ssem[b], add=True)

    def step(ci, s, first=False):
        s1 = (s + 1) % 3
        s2 = (s + 2) % 3

        @pl.when(ci + 2 < NCHUNK)
        def _():
            pltpu.async_copy(src_hbm.at[wid, ci + 2], sidx[s2], isem[s2])

        @pl.when(ci + 1 < NCHUNK)
        def _():
            if not first:
                wait_scatter(s1)
            pltpu.make_async_copy(src_hbm.at[wid, 0], sidx[s1], isem[s1]).wait()
            start_gather(ci + 1, s1)

        wait_gather(s)
        compute(ci, s)

    pltpu.sync_copy(src_hbm.at[wid, 0], sidx[0])
    pltpu.async_copy(src_hbm.at[wid, 1], sidx[1], isem[1])
    start_gather(0, 0)
    step(0, 0, first=True)
    step(1, 1, first=True)

    @pl.loop(0, (NCHUNK - 2) // 3)
    def _(p):
        step(3 * p + 2, 2)
        step(3 * p + 3, 0)
        step(3 * p + 4, 1)

    wait_scatter(0)
    wait_scatter(1)

    plsc.subcore_barrier()

    pltpu.sync_copy(acc_sh.at[pl.ds(row0, RPS)], acc_hbm.at[c_idx, pl.ds(row0, RPS)])
    pltpu.sync_copy(den_sh.at[pl.ds(row0, RPS)], den_hbm.at[c_idx, pl.ds(row0, RPS)])


_sc_edge = pl.kernel(
    _sc_edge_body,
    out_type=(jax.ShapeDtypeStruct((NC, NP_, D), jnp.float32),
              jax.ShapeDtypeStruct((NC, NP_), jnp.float32)),
    mesh=_mesh,
    compiler_params=pltpu.CompilerParams(needs_layout_passes=False,
                                         use_tc_tiling_on_sc=False),
    scratch_types=[
        pltpu.VMEM((NCHUNK, C), jnp.int32),
        (pltpu.VMEM((C,), jnp.int32),) * 3,
        (pltpu.VMEM((C, 2 * ATT), jnp.float32),) * 3,
        (pltpu.VMEM((C, 2 * ATT), jnp.float32),) * 3,
        (pltpu.VMEM((C, D), jnp.float32),) * 3,
        (pltpu.VMEM((C,), jnp.float32),) * 3,
        pltpu.VMEM((RPS,), jnp.float32),
        (pltpu.SemaphoreType.DMA,) * 3,
        (pltpu.SemaphoreType.DMA,) * 3,
        (pltpu.SemaphoreType.DMA,) * 3,
        pltpu.VMEM_SHARED((NP_, D), jnp.float32),
        pltpu.VMEM_SHARED((NP_,), jnp.float32),
    ],
)


BN = 1000


def _mm_body(x_ref, wqk_ref, wv_ref, qk_ref, v_ref):
    xb = x_ref[...]
    qk_ref[...] = jnp.dot(xb, wqk_ref[...], preferred_element_type=jnp.float32)
    v_ref[...] = jnp.dot(xb, wv_ref[...], preferred_element_type=jnp.float32)


def _mm(x, wqk, wv):
    return pl.pallas_call(
        _mm_body,
        grid=(N // BN,),
        in_specs=[pl.BlockSpec((BN, D), lambda i: (i, 0)),
                  pl.BlockSpec((D, 2 * ATT), lambda i: (0, 0)),
                  pl.BlockSpec((D, D), lambda i: (0, 0))],
        out_specs=[pl.BlockSpec((BN, 2 * ATT), lambda i: (i, 0)),
                   pl.BlockSpec((BN, D), lambda i: (i, 0))],
        out_shape=(jax.ShapeDtypeStruct((N, 2 * ATT), jnp.float32),
                   jax.ShapeDtypeStruct((N, D), jnp.float32)),
    )(x, wqk, wv)


def _combine_mm_body(acc_ref, den_ref, wqk_ref, wv_ref, qk_ref, v_ref):
    a = acc_ref[0] + acc_ref[1]
    d = den_ref[0] + den_ref[1] + 1e-16
    h = a / d
    h = jnp.where(h >= 0.0, h, 0.1 * h)
    qk_ref[...] = jnp.dot(h, wqk_ref[...], preferred_element_type=jnp.float32)
    v_ref[...] = jnp.dot(h, wv_ref[...], preferred_element_type=jnp.float32)


def _combine_mm(acc, den, wqk, wv):
    return pl.pallas_call(
        _combine_mm_body,
        grid=(N // BN,),
        in_specs=[pl.BlockSpec((NC, BN, D), lambda i: (0, i, 0)),
                  pl.BlockSpec((NC, BN, 1), lambda i: (0, i, 0)),
                  pl.BlockSpec((D, 2 * ATT), lambda i: (0, 0)),
                  pl.BlockSpec((D, D), lambda i: (0, 0))],
        out_specs=[pl.BlockSpec((BN, 2 * ATT), lambda i: (i, 0)),
                   pl.BlockSpec((BN, D), lambda i: (i, 0))],
        out_shape=(jax.ShapeDtypeStruct((N, 2 * ATT), jnp.float32),
                   jax.ShapeDtypeStruct((N, D), jnp.float32)),
    )(acc, den, wqk, wv)


def _final_body(acc_ref, den_ref, out_ref):
    a = acc_ref[0] + acc_ref[1]
    d = den_ref[0] + den_ref[1] + 1e-16
    h = a / d
    out_ref[...] = jnp.where(h >= 0.0, h, 0.1 * h)


def _final(acc, den):
    return pl.pallas_call(
        _final_body,
        grid=(N // BN,),
        in_specs=[pl.BlockSpec((NC, BN, D), lambda i: (0, i, 0)),
                  pl.BlockSpec((NC, BN, 1), lambda i: (0, i, 0))],
        out_specs=pl.BlockSpec((BN, D), lambda i: (i, 0)),
        out_shape=jax.ShapeDtypeStruct((N, D), jnp.float32),
    )(acc, den)


def kernel(x, edge_index, Wq1, Wk1, Wv1, Wq2, Wk2, Wv2):
    src3 = edge_index[0].reshape(NW, NCHUNK, C)
    dst3 = edge_index[1].reshape(NW, NCHUNK, C)

    wqk1 = jnp.concatenate([Wq1, Wk1], axis=1)
    wqk2 = jnp.concatenate([Wq2, Wk2], axis=1)

    qk1, v1 = _mm(x, wqk1, Wv1)
    acc1, den1 = _sc_edge(src3, dst3, qk1, v1)
    qk2, v2 = _combine_mm(acc1, den1[..., None], wqk2, Wv2)
    acc2, den2 = _sc_edge(src3, dst3, qk2, v2)
    return _final(acc2, den2[..., None])
